# baseline (speedup 1.0000x reference)
import jax
import jax.numpy as jnp
from jax import lax
from jax.experimental import pallas as pl
from jax.experimental.pallas import tpu as pltpu
from jax.experimental.pallas import tpu_sc as plsc

BATCH = 16384
D = 128
L = 16
NC = 2
NS = 16
NW = NC * NS
ROWS_PER_W = BATCH // NW
CHUNK = 128
NCHUNK = ROWS_PER_W // CHUNK
HALF = CHUNK // 2
NSPLIT = ROWS_PER_W // HALF


def _neg_sqrt(s):
    sc = jnp.maximum(s, jnp.float32(1e-30))
    ix = lax.bitcast_convert_type(sc, jnp.int32)
    iy = jnp.int32(0x5F3759DF) - lax.shift_right_arithmetic(ix, 1)
    y = lax.bitcast_convert_type(iy, jnp.float32)
    half = jnp.float32(0.5) * sc
    for _ in range(3):
        y = y * (jnp.float32(1.5) - half * y * y)
    return -(sc * y)


def _sc_body(hidx_hbm, ridx_hbm, tidx_hbm, node_hbm, re_hbm, out_hbm,
             hidx_v, ridx_v, tidx_v,
             hb0, tb0, hb1, tb1,
             accs_v, shared_v, y_v, out_v,
             isem, hsem0, tsem0, rsem0, hsem1, tsem1, rsem1, csem):
    wid = lax.axis_index("s") * NC + lax.axis_index("c")
    sid = lax.axis_index("s")
    wbase = wid * ROWS_PER_W

    wsl = pl.ds(wbase, ROWS_PER_W)
    di = pltpu.async_copy(hidx_hbm.at[wsl], hidx_v, isem)
    dr = pltpu.async_copy(ridx_hbm.at[wsl], ridx_v, isem)
    dt = pltpu.async_copy(tidx_hbm.at[wsl], tidx_v, isem)
    di.wait()
    dr.wait()
    dt.wait()

    bufs = ((hb0, tb0, hsem0, tsem0, rsem0), (hb1, tb1, hsem1, tsem1, rsem1))

    def start_ht(c):
        hb, tb, hsem, tsem2, _ = bufs[c % 2]
        csl = pl.ds(c * CHUNK, CHUNK)
        return (pltpu.async_copy(node_hbm.at[hidx_v.at[csl]], hb, hsem),
                pltpu.async_copy(node_hbm.at[tidx_v.at[csl]], tb, tsem2))

    def start_radd(c):
        hb, _, _, _, rsem = bufs[c % 2]
        csl = pl.ds(c * CHUNK, CHUNK)
        return pltpu.async_copy(re_hbm.at[ridx_v.at[csl]], hb, rsem, add=True)

    cols = []
    dh, dt = start_ht(0)
    dh.wait()
    radd = {0: start_radd(0)}
    pend_ht = {0: (None, dt)}
    if NCHUNK > 1:
        pend_ht[1] = start_ht(1)
    for c in range(NCHUNK):
        hb, tb, _, _, _ = bufs[c % 2]
        radd[c].wait()
        pend_ht[c][1].wait()
        if c + 1 < NCHUNK:
            pend_ht[c + 1][0].wait()
            radd[c + 1] = start_radd(c + 1)

        def row(i, _):
            acc = jnp.zeros((L,), jnp.float32)
            for j in range(D // L):
                sl = pl.ds(j * L, L)
                d = hb[i, sl] - tb[i, sl]
                acc = acc + d * d
            accs_v[i, pl.ds(0, L)] = acc
            s = acc
            for off in (8, 4, 2, 1):
                s = s + accs_v[i, pl.ds(off, L)]
                accs_v[i, pl.ds(0, L)] = s
            return 0

        lax.fori_loop(0, CHUNK, row, 0)

        cols.append(pltpu.async_copy(accs_v.at[:, 0],
                                     shared_v.at[sid, pl.ds(c * CHUNK, CHUNK)],
                                     csem))
        if c + 2 < NCHUNK:
            pend_ht[c + 2] = start_ht(c + 2)

    for d in cols:
        d.wait()
    pltpu.sync_copy(shared_v.at[sid], y_v)
    for g in range(ROWS_PER_W // L):
        sl = pl.ds(g * L, L)
        out_v[sl] = _neg_sqrt(y_v[sl])
    pltpu.sync_copy(out_v, out_hbm.at[wsl])


@jax.jit
def _run(hidx, ridx, tidx, node_embedding, node_re_embedding):
    mesh = plsc.VectorSubcoreMesh(core_axis_name="c", subcore_axis_name="s")
    return pl.kernel(
        _sc_body,
        out_type=jax.ShapeDtypeStruct((BATCH,), jnp.float32),
        mesh=mesh,
        scratch_types=[
            pltpu.VMEM((ROWS_PER_W,), jnp.int32),
            pltpu.VMEM((ROWS_PER_W,), jnp.int32),
            pltpu.VMEM((ROWS_PER_W,), jnp.int32),
            pltpu.VMEM((CHUNK, D), jnp.float32),
            pltpu.VMEM((CHUNK, D), jnp.float32),
            pltpu.VMEM((CHUNK, D), jnp.float32),
            pltpu.VMEM((CHUNK, D), jnp.float32),
            pltpu.VMEM((CHUNK, 2 * L), jnp.float32),
            pltpu.VMEM_SHARED((NS, ROWS_PER_W), jnp.float32),
            pltpu.VMEM((ROWS_PER_W,), jnp.float32),
            pltpu.VMEM((ROWS_PER_W,), jnp.float32),
            pltpu.SemaphoreType.DMA,
            pltpu.SemaphoreType.DMA,
            pltpu.SemaphoreType.DMA,
            pltpu.SemaphoreType.DMA,
            pltpu.SemaphoreType.DMA,
            pltpu.SemaphoreType.DMA,
            pltpu.SemaphoreType.DMA,
            pltpu.SemaphoreType.DMA,
        ],
    )(hidx, ridx, tidx, node_embedding, node_re_embedding).reshape(BATCH, 1)


def kernel(sample, node_embedding, node_re_embedding):
    sample = sample.astype(jnp.int32)
    return _run(sample[:, 0], sample[:, 1], sample[:, 2],
                node_embedding, node_re_embedding)


# speedup vs baseline: 1.1568x; 1.1568x over previous
import jax
import jax.numpy as jnp
from jax import lax
from jax.experimental import pallas as pl
from jax.experimental.pallas import tpu as pltpu
from jax.experimental.pallas import tpu_sc as plsc

BATCH = 16384
D = 128
L = 16
NC = 2
NS = 16
NW = NC * NS
ROWS_PER_W = BATCH // NW
CHUNK = 128
NCHUNK = ROWS_PER_W // CHUNK
HALF = CHUNK // 2
NSPLIT = ROWS_PER_W // HALF


def _neg_sqrt(s):
    sc = jnp.maximum(s, jnp.float32(1e-30))
    ix = lax.bitcast_convert_type(sc, jnp.int32)
    iy = jnp.int32(0x5F3759DF) - lax.shift_right_arithmetic(ix, 1)
    y = lax.bitcast_convert_type(iy, jnp.float32)
    half = jnp.float32(0.5) * sc
    for _ in range(3):
        y = y * (jnp.float32(1.5) - half * y * y)
    return -(sc * y)


def _sc_body(hidx_hbm, ridx_hbm, tidx_hbm, node_hbm, re_hbm, out_hbm,
             hidx_v, ridx_v, tidx_v,
             hb0, rb0, tb0, hb1, rb1, tb1,
             accs_v, shared_v, y_v, out_v,
             isem, gsem0, gsem1, tsem):
    wid = lax.axis_index("s") * NC + lax.axis_index("c")
    sid = lax.axis_index("s")
    wbase = wid * ROWS_PER_W

    wsl = pl.ds(wbase, ROWS_PER_W)
    di = pltpu.async_copy(hidx_hbm.at[wsl], hidx_v, isem)
    dr = pltpu.async_copy(ridx_hbm.at[wsl], ridx_v, isem)
    dt = pltpu.async_copy(tidx_hbm.at[wsl], tidx_v, isem)
    di.wait()
    dr.wait()
    dt.wait()

    bufs = ((hb0, rb0, tb0, gsem0), (hb1, rb1, tb1, gsem1))

    def start(c):
        hb, rb, tb, sem = bufs[c % 2]
        csl = pl.ds(c * CHUNK, CHUNK)
        return (pltpu.async_copy(node_hbm.at[hidx_v.at[csl]], hb, sem),
                pltpu.async_copy(re_hbm.at[ridx_v.at[csl]], rb, sem),
                pltpu.async_copy(node_hbm.at[tidx_v.at[csl]], tb, sem))

    cols = []
    pending = start(0)
    for c in range(NCHUNK):
        hb, rb, tb, _ = bufs[c % 2]
        for d in pending:
            d.wait()
        if c + 1 < NCHUNK:
            pending = start(c + 1)

        def row(i, _):
            acc = jnp.zeros((L,), jnp.float32)
            for j in range(D // L):
                sl = pl.ds(j * L, L)
                d = hb[i, sl] + rb[i, sl] - tb[i, sl]
                acc = acc + d * d
            accs_v[i, pl.ds(0, L)] = acc
            s = acc
            for off in (8, 4, 2, 1):
                s = s + accs_v[i, pl.ds(off, L)]
                accs_v[i, pl.ds(0, L)] = s
            return 0

        lax.fori_loop(0, CHUNK, row, 0)

        cols.append(pltpu.async_copy(accs_v.at[:, 0],
                                     shared_v.at[sid, pl.ds(c * CHUNK, CHUNK)],
                                     tsem))

    for d in cols:
        d.wait()
    pltpu.sync_copy(shared_v.at[sid], y_v)
    for g in range(ROWS_PER_W // L):
        sl = pl.ds(g * L, L)
        out_v[sl] = _neg_sqrt(y_v[sl])
    pltpu.sync_copy(out_v, out_hbm.at[wsl])


@jax.jit
def _run(hidx, ridx, tidx, node_embedding, node_re_embedding):
    mesh = plsc.VectorSubcoreMesh(core_axis_name="c", subcore_axis_name="s")
    return pl.kernel(
        _sc_body,
        out_type=jax.ShapeDtypeStruct((BATCH,), jnp.float32),
        mesh=mesh,
        scratch_types=[
            pltpu.VMEM((ROWS_PER_W,), jnp.int32),
            pltpu.VMEM((ROWS_PER_W,), jnp.int32),
            pltpu.VMEM((ROWS_PER_W,), jnp.int32),
            pltpu.VMEM((CHUNK, D), jnp.float32),
            pltpu.VMEM((CHUNK, D), jnp.float32),
            pltpu.VMEM((CHUNK, D), jnp.float32),
            pltpu.VMEM((CHUNK, D), jnp.float32),
            pltpu.VMEM((CHUNK, D), jnp.float32),
            pltpu.VMEM((CHUNK, D), jnp.float32),
            pltpu.VMEM((CHUNK, 2 * L), jnp.float32),
            pltpu.VMEM_SHARED((NS, ROWS_PER_W), jnp.float32),
            pltpu.VMEM((ROWS_PER_W,), jnp.float32),
            pltpu.VMEM((ROWS_PER_W,), jnp.float32),
            pltpu.SemaphoreType.DMA,
            pltpu.SemaphoreType.DMA,
            pltpu.SemaphoreType.DMA,
            pltpu.SemaphoreType.DMA,
        ],
    )(hidx, ridx, tidx, node_embedding, node_re_embedding).reshape(BATCH, 1)


def kernel(sample, node_embedding, node_re_embedding):
    sample = sample.astype(jnp.int32)
    return _run(sample[:, 0], sample[:, 1], sample[:, 2],
                node_embedding, node_re_embedding)
